# baseline (speedup 1.0000x reference)
import jax
import jax.numpy as jnp
from jax import lax
from jax.experimental import pallas as pl
from jax.experimental.pallas import tpu as pltpu
from jax.experimental.pallas import tpu_sc as plsc

_NC = 2
_NS = 16
_CH = 128
_BBLK = 1024
_PACK = 4


def _body(tab_hbm, idx_hbm, out_hbm,
          idx_v, gq_v, buf_a, buf_b, slab_v, sem_g, sem_i, sem_w):
    n_fields, n_bblk = idx_hbm.shape[0], idx_hbm.shape[1]
    d = 128 // _PACK
    rpf = tab_hbm.shape[0] // n_fields
    n_ch = _BBLK // _CH
    units2 = n_fields * n_bblk // (_NC * _NS)
    c = lax.axis_index("c")
    s = lax.axis_index("s")
    iota16 = lax.iota(jnp.int32, 16)

    def p2_idx(i, p, issue):
        u = (s * _NC + c) * units2 + i
        src = idx_hbm.at[u // n_bblk, u % n_bblk]
        if issue:
            pltpu.async_copy(src, idx_v.at[p], sem_i.at[p])
        else:
            pltpu.make_async_copy(src, idx_v.at[p], sem_i.at[p]).wait()

    def p2_slab(i, issue):
        u = (s * _NC + c) * units2 + i
        dst = out_hbm.at[pl.ds((u // n_bblk) * d, d),
                         pl.ds((u % n_bblk) * _BBLK, _BBLK)]
        if issue:
            pltpu.async_copy(slab_v, dst, sem_w)
        else:
            pltpu.make_async_copy(slab_v, dst, sem_w).wait()

    def p2_gather(buf, pg, issue):
        src = tab_hbm.at[gq_v.at[pg, 0]]
        if issue:
            pltpu.async_copy(src, buf, sem_g.at[pg])
        else:
            pltpu.make_async_copy(src, buf, sem_g.at[pg]).wait()

    def p2_unit(i, carry):
        p = i % 2
        p2_idx(i, p, False)

        @pl.when(i + 1 < units2)
        def _():
            p2_idx(i + 1, (i + 1) % 2, True)

        @pl.when(i >= 1)
        def _():
            p2_slab(i - 1, False)

        u = (s * _NC + c) * units2 + i
        fbase = (u // n_bblk) * rpf

        def gq_compute(ch, pg):
            for j in range(_CH // 16):
                iv = idx_v[p, ch, pl.ds(16 * j, 16)]
                gq_v[pg, 0, pl.ds(16 * j, 16)] = (iv >> 2) + fbase
                gq_v[pg, 1, pl.ds(16 * j, 16)] = (iv & (_PACK - 1)) * d

        def extract(buf, ch, pg):
            for j in range(_CH // 16):
                q32 = gq_v[pg, 1, pl.ds(16 * j, 16)]
                b_idx = iota16 + 16 * j
                for dd in range(d):
                    vals = plsc.load_gather(buf, [b_idx, q32 + dd])
                    slab_v[dd, pl.ds(ch * _CH + 16 * j, 16)] = vals

        gq_compute(0, 0)
        p2_gather(buf_a, 0, True)

        def chunk_pair(m, cr):
            ch = 2 * m
            gq_compute(ch + 1, 1)
            p2_gather(buf_b, 1, True)
            p2_gather(buf_a, 0, False)
            extract(buf_a, ch, 0)
            @pl.when(ch + 2 < n_ch)
            def _():
                gq_compute(ch + 2, 0)
                p2_gather(buf_a, 0, True)

            p2_gather(buf_b, 1, False)
            extract(buf_b, ch + 1, 1)
            return cr

        lax.fori_loop(0, n_ch // 2, chunk_pair, 0)
        p2_slab(i, True)
        return carry

    p2_idx(0, 0, True)
    lax.fori_loop(0, units2, p2_unit, 0)
    p2_slab(units2 - 1, False)


def kernel(metas, tables):
    f, b = metas.shape
    v, d = tables.shape[1], tables.shape[2]
    n_bblk = b // _BBLK

    idx = metas.astype(jnp.int32).reshape(f, n_bblk, _BBLK // _CH, _CH)
    tabp = lax.bitcast_convert_type(
        lax.bitcast_convert_type(tables, jnp.int32).reshape(
            f * v // _PACK, _PACK * d),
        jnp.float32,
    )

    run = pl.kernel(
        _body,
        out_type=jax.ShapeDtypeStruct((f * d, b), jnp.float32),
        mesh=plsc.VectorSubcoreMesh(core_axis_name="c", subcore_axis_name="s"),
        scratch_types=[
            pltpu.VMEM((2, _BBLK // _CH, _CH), jnp.int32),
            pltpu.VMEM((2, 2, _CH), jnp.int32),
            pltpu.VMEM((_CH, _PACK * d), jnp.float32),
            pltpu.VMEM((_CH, _PACK * d), jnp.float32),
            pltpu.VMEM((d, _BBLK), jnp.float32),
            pltpu.SemaphoreType.DMA((2,)),
            pltpu.SemaphoreType.DMA((2,)),
            pltpu.SemaphoreType.DMA,
        ],
        compiler_params=pltpu.CompilerParams(
            use_tc_tiling_on_sc=True, needs_layout_passes=False
        ),
    )
    return run(tabp, idx).T


# speedup vs baseline: 1.1760x; 1.1760x over previous
import jax
import jax.numpy as jnp
from jax import lax
from jax.experimental import pallas as pl
from jax.experimental.pallas import tpu as pltpu
from jax.experimental.pallas import tpu_sc as plsc

_NC = 2
_NS = 16
_NW = _NC * _NS
_CH = 128


def _body(tab_hbm, idx_hbm, out_hbm, idx_v, rows_v, sem):
    n_fields, n_chunks, _ = idx_v.shape
    bpw, d = rows_v.shape
    w = lax.axis_index("s") * _NC + lax.axis_index("c")
    pltpu.sync_copy(idx_hbm.at[w], idx_v)
    base = w * bpw

    def field_step(f, carry):
        cps = [
            pltpu.async_copy(
                tab_hbm.at[idx_v.at[f, c]],
                rows_v.at[pl.ds(c * _CH, _CH)],
                sem,
            )
            for c in range(n_chunks)
        ]
        for cp in cps:
            cp.wait()
        pltpu.sync_copy(
            rows_v, out_hbm.at[pl.ds(base, bpw), pl.ds(f * d, d)]
        )
        return carry

    lax.fori_loop(0, n_fields, field_step, 0)


def kernel(metas, tables):
    f, b = metas.shape
    v, d = tables.shape[1], tables.shape[2]
    bpw = b // _NW
    n_chunks = bpw // _CH

    idx = metas.astype(jnp.int32) + (jnp.arange(f, dtype=jnp.int32) * v)[:, None]
    idx = idx.reshape(f, _NW, n_chunks, _CH).transpose(1, 0, 2, 3)
    tab = tables.reshape(f * v, d)

    run = pl.kernel(
        _body,
        out_type=jax.ShapeDtypeStruct((b, f * d), jnp.float32),
        mesh=plsc.VectorSubcoreMesh(core_axis_name="c", subcore_axis_name="s"),
        scratch_types=[
            pltpu.VMEM((f, n_chunks, _CH), jnp.int32),
            pltpu.VMEM((bpw, d), jnp.float32),
            pltpu.SemaphoreType.DMA,
        ],
        compiler_params=pltpu.CompilerParams(use_tc_tiling_on_sc=False),
    )
    return run(tab, idx)
